# baseline (speedup 1.0000x reference)
import functools

import jax
import jax.numpy as jnp
from jax import lax
from jax.experimental import pallas as pl
from jax.experimental.pallas import tpu as pltpu
from jax.experimental.pallas import tpu_sc as plsc

_B = 16384
_NUM_NUM = 13
_NUM_FIELDS = 26
_VOCAB = 100000
_EMB = 64
_H1 = 1024
_H2 = 512
_EPS = 1e-5

_NC = 2
_NS = 16
_NW = _NC * _NS
_CH = 128
_NBUF = 4

_GROUPS = (7, 7, 6, 6)

_BLK = 512


def _gather_body(nch, perw, idx_hbm, tab_hbm, out_hbm, idx_v, rows_v, gsem):
  wid = lax.axis_index("s") * _NC + lax.axis_index("c")
  pltpu.sync_copy(idx_hbm.at[pl.ds(wid * nch, nch)], idx_v)
  base = wid * perw

  @pl.loop(0, nch, step=_NBUF)
  def _grp(g):
    descs = []
    for b in range(_NBUF):
      descs.append(
          pltpu.async_copy(tab_hbm.at[idx_v.at[g + b]], rows_v.at[b], gsem))
    for d in descs:
      d.wait()
    for b in range(_NBUF):
      pltpu.sync_copy(rows_v.at[b],
                      out_hbm.at[pl.ds(base + (g + b) * _CH, _CH)])


def _make_gather(nf):
  tot = _B * nf
  perw = tot // _NW
  nch = perw // _CH
  mesh = plsc.VectorSubcoreMesh(
      core_axis_name="c", subcore_axis_name="s",
      num_cores=_NC, num_subcores=_NS)
  return pl.kernel(
      functools.partial(_gather_body, nch, perw),
      out_type=jax.ShapeDtypeStruct((tot, _EMB), jnp.float32),
      mesh=mesh,
      scratch_types=[
          pltpu.VMEM((nch, _CH), jnp.int32),
          pltpu.VMEM((_NBUF, _CH, _EMB), jnp.float32),
          pltpu.SemaphoreType.DMA,
      ],
      compiler_params=pltpu.CompilerParams(use_tc_tiling_on_sc=False),
  )


def _mlp_block(xn_ref, xe0_ref, xe1_ref, xe2_ref, xe3_ref, w1a_ref,
               w10_ref, w11_ref, w12_ref, w13_ref, b1_ref, g1_ref, be1_ref,
               m1_ref, v1_ref, w2_ref, b2_ref, g2_ref, be2_ref, m2_ref,
               v2_ref, w3_ref, b3_ref, out_ref):
  s1 = g1_ref[...] * lax.rsqrt(v1_ref[...] + _EPS)
  t1 = be1_ref[...] - m1_ref[...] * s1
  s2 = g2_ref[...] * lax.rsqrt(v2_ref[...] + _EPS)
  t2 = be2_ref[...] - m2_ref[...] * s2

  h = jnp.dot(xn_ref[...], w1a_ref[...], preferred_element_type=jnp.float32)
  for xe, w in ((xe0_ref, w10_ref), (xe1_ref, w11_ref),
                (xe2_ref, w12_ref), (xe3_ref, w13_ref)):
    h = h + jnp.dot(xe[...], w[...], preferred_element_type=jnp.float32)
  h = jnp.maximum(h + b1_ref[...], 0.0) * s1 + t1
  h = jnp.dot(h, w2_ref[...], preferred_element_type=jnp.float32)
  h = jnp.maximum(h + b2_ref[...], 0.0) * s2 + t2
  out_ref[...] = (
      jnp.dot(h, w3_ref[...], preferred_element_type=jnp.float32)
      + b3_ref[...])


def _mlp(x_num, embs, w1a, w1parts, b1, g1, be1, m1, v1, W2, b2, g2, be2,
         m2, v2, W3, b3):
  grid = (_B // _BLK,)
  row = lambda i: (i, 0)
  rep = lambda i: (0, 0)
  emb_specs = [pl.BlockSpec((_BLK, nf * _EMB), row) for nf in _GROUPS]
  w1_specs = [pl.BlockSpec((nf * _EMB, _H1), rep) for nf in _GROUPS]
  return pl.pallas_call(
      _mlp_block,
      grid=grid,
      in_specs=[
          pl.BlockSpec((_BLK, _NUM_NUM), row),
          *emb_specs,
          pl.BlockSpec((_NUM_NUM, _H1), rep),
          *w1_specs,
          pl.BlockSpec((1, _H1), rep),
          pl.BlockSpec((1, _H1), rep),
          pl.BlockSpec((1, _H1), rep),
          pl.BlockSpec((1, _H1), rep),
          pl.BlockSpec((1, _H1), rep),
          pl.BlockSpec((_H1, _H2), rep),
          pl.BlockSpec((1, _H2), rep),
          pl.BlockSpec((1, _H2), rep),
          pl.BlockSpec((1, _H2), rep),
          pl.BlockSpec((1, _H2), rep),
          pl.BlockSpec((1, _H2), rep),
          pl.BlockSpec((_H2, 1), rep),
          pl.BlockSpec((1, 1), rep),
      ],
      out_specs=pl.BlockSpec((_BLK, 1), row),
      out_shape=jax.ShapeDtypeStruct((_B, 1), jnp.float32),
  )(x_num, *embs, w1a, *w1parts, b1, g1, be1, m1, v1, W2, b2, g2, be2, m2,
    v2, W3, b3)


def kernel(x_num, x_cat, tables, W1, b1, g1, be1, m1, v1, W2, b2, g2, be2,
           m2, v2, W3, b3):
  embs = []
  w1parts = []
  f0 = 0
  for nf in _GROUPS:
    f1 = f0 + nf
    tabg = tables[f0:f1].reshape(nf * _VOCAB, _EMB)
    offs = (jnp.arange(nf, dtype=jnp.int32) * _VOCAB)[None, :]
    idx2d = (x_cat[:, f0:f1] + offs).reshape(_B * nf // _CH, _CH)
    e = _make_gather(nf)(idx2d, tabg)
    embs.append(e.reshape(_B, nf * _EMB))
    w1parts.append(W1[_NUM_NUM + f0 * _EMB:_NUM_NUM + f1 * _EMB])
    f0 = f1

  r = lambda a: a.reshape(1, -1)
  out = _mlp(x_num, embs, W1[:_NUM_NUM], w1parts, r(b1), r(g1), r(be1),
             r(m1), r(v1), W2, r(b2), r(g2), r(be2), r(m2), r(v2), W3,
             r(b3))
  return out[:, 0]


# speedup vs baseline: 1.3313x; 1.3313x over previous
import jax
import jax.numpy as jnp
from jax import lax
from jax.experimental import pallas as pl
from jax.experimental.pallas import tpu as pltpu
from jax.experimental.pallas import tpu_sc as plsc

_B = 16384
_NUM_NUM = 13
_NUM_FIELDS = 26
_VOCAB = 100000
_EMB = 64
_H1 = 1024
_H2 = 512
_EPS = 1e-5

_NC = 2
_NS = 16
_NW = _NC * _NS
_TOT = _B * _NUM_FIELDS
_PERW = _TOT // _NW
_CH = 128
_NCH = _PERW // _CH
_CPF = _B // _CH
_NBUF = 8

_BLK = 512


def _gather_body(idx_hbm, tab_hbm, out_hbm, idx_v, rows_v, gsem):
  wid = lax.axis_index("s") * _NC + lax.axis_index("c")
  pltpu.sync_copy(idx_hbm.at[pl.ds(wid * _NCH, _NCH)], idx_v)
  base = wid * _PERW
  g0 = wid * _NCH

  @pl.loop(0, _NCH, step=_NBUF)
  def _grp(g):
    descs = []
    for b in range(_NBUF):
      f = (g0 + g + b) // _CPF
      descs.append(
          pltpu.async_copy(tab_hbm.at[f].at[idx_v.at[g + b]], rows_v.at[b],
                           gsem))
    for d in descs:
      d.wait()
    for b in range(_NBUF):
      pltpu.sync_copy(rows_v.at[b],
                      out_hbm.at[pl.ds(base + (g + b) * _CH, _CH)])


def _make_gather():
  mesh = plsc.VectorSubcoreMesh(
      core_axis_name="c", subcore_axis_name="s",
      num_cores=_NC, num_subcores=_NS)
  return pl.kernel(
      _gather_body,
      out_type=jax.ShapeDtypeStruct((_TOT, _EMB), jnp.float32),
      mesh=mesh,
      scratch_types=[
          pltpu.VMEM((_NCH, _CH), jnp.int32),
          pltpu.VMEM((_NBUF, _CH, _EMB), jnp.float32),
          pltpu.SemaphoreType.DMA,
      ],
      compiler_params=pltpu.CompilerParams(use_tc_tiling_on_sc=False),
  )


def _mlp_block(xn_ref, xe_ref, w1a_ref, w1b_ref, b1_ref, g1_ref, be1_ref,
               m1_ref, v1_ref, w2_ref, b2_ref, g2_ref, be2_ref, m2_ref,
               v2_ref, w3_ref, b3_ref, out_ref):
  s1 = g1_ref[...] * lax.rsqrt(v1_ref[...] + _EPS)
  t1 = be1_ref[...] - m1_ref[...] * s1
  s2 = g2_ref[...] * lax.rsqrt(v2_ref[...] + _EPS)
  t2 = be2_ref[...] - m2_ref[...] * s2

  h = jnp.dot(xn_ref[...], w1a_ref[...], preferred_element_type=jnp.float32)
  h = h + jnp.dot(xe_ref[...], w1b_ref[...],
                  preferred_element_type=jnp.float32)
  h = jnp.maximum(h + b1_ref[...], 0.0) * s1 + t1
  h = jnp.dot(h, w2_ref[...], preferred_element_type=jnp.float32)
  h = jnp.maximum(h + b2_ref[...], 0.0) * s2 + t2
  out_ref[...] = (
      jnp.dot(h, w3_ref[...], preferred_element_type=jnp.float32)
      + b3_ref[...])


def _mlp(x_num, embs, W1a, W1b, b1, g1, be1, m1, v1, W2, b2, g2, be2, m2,
         v2, W3, b3):
  grid = (_B // _BLK,)
  row = lambda i: (i, 0)
  rep = lambda i: (0, 0)
  return pl.pallas_call(
      _mlp_block,
      grid=grid,
      in_specs=[
          pl.BlockSpec((_BLK, _NUM_NUM), row),
          pl.BlockSpec((_BLK, _NUM_FIELDS * _EMB), row),
          pl.BlockSpec((_NUM_NUM, _H1), rep),
          pl.BlockSpec((_NUM_FIELDS * _EMB, _H1), rep),
          pl.BlockSpec((1, _H1), rep),
          pl.BlockSpec((1, _H1), rep),
          pl.BlockSpec((1, _H1), rep),
          pl.BlockSpec((1, _H1), rep),
          pl.BlockSpec((1, _H1), rep),
          pl.BlockSpec((_H1, _H2), rep),
          pl.BlockSpec((1, _H2), rep),
          pl.BlockSpec((1, _H2), rep),
          pl.BlockSpec((1, _H2), rep),
          pl.BlockSpec((1, _H2), rep),
          pl.BlockSpec((1, _H2), rep),
          pl.BlockSpec((_H2, 1), rep),
          pl.BlockSpec((1, 1), rep),
      ],
      out_specs=pl.BlockSpec((_BLK, 1), row),
      out_shape=jax.ShapeDtypeStruct((_B, 1), jnp.float32),
  )(x_num, embs, W1a, W1b, b1, g1, be1, m1, v1, W2, b2, g2, be2, m2, v2,
    W3, b3)


def kernel(x_num, x_cat, tables, W1, b1, g1, be1, m1, v1, W2, b2, g2, be2,
           m2, v2, W3, b3):
  idx2d = x_cat.T.reshape(_NUM_FIELDS * _CPF, _CH)
  embs = _make_gather()(idx2d, tables)
  embs = embs.reshape(_NUM_FIELDS, _B, _EMB).transpose(1, 0, 2)
  embs = embs.reshape(_B, _NUM_FIELDS * _EMB)

  r = lambda a: a.reshape(1, -1)
  out = _mlp(x_num, embs, W1[:_NUM_NUM], W1[_NUM_NUM:], r(b1), r(g1),
             r(be1), r(m1), r(v1), W2, r(b2), r(g2), r(be2), r(m2), r(v2),
             W3, r(b3))
  return out[:, 0]


# speedup vs baseline: 1.5021x; 1.1284x over previous
import jax
import jax.numpy as jnp
from jax import lax
from jax.experimental import pallas as pl
from jax.experimental.pallas import tpu as pltpu
from jax.experimental.pallas import tpu_sc as plsc

_B = 16384
_NUM_NUM = 13
_NUM_FIELDS = 26
_VOCAB = 100000
_EMB = 64
_H1 = 1024
_H2 = 512
_EPS = 1e-5

_NC = 2
_NS = 16
_NW = _NC * _NS
_TOT = _B * _NUM_FIELDS
_PERW = _TOT // _NW
_CH = 128
_NCH = _PERW // _CH
_NBUF = 8

_BLK = 512


def _gather_body(idx_hbm, tab_hbm, out_hbm, idx_v, rows_v, gsem):
  wid = lax.axis_index("s") * _NC + lax.axis_index("c")
  pltpu.sync_copy(idx_hbm.at[pl.ds(wid * _NCH, _NCH)], idx_v)
  base = wid * _PERW

  @pl.loop(0, _NCH, step=_NBUF)
  def _grp(g):
    descs = []
    for b in range(_NBUF):
      descs.append(
          pltpu.async_copy(tab_hbm.at[idx_v.at[g + b]], rows_v.at[b], gsem))
    for d in descs:
      d.wait()
    for b in range(_NBUF):
      pltpu.sync_copy(rows_v.at[b],
                      out_hbm.at[pl.ds(base + (g + b) * _CH, _CH)])


def _make_gather():
  mesh = plsc.VectorSubcoreMesh(
      core_axis_name="c", subcore_axis_name="s",
      num_cores=_NC, num_subcores=_NS)
  return pl.kernel(
      _gather_body,
      out_type=jax.ShapeDtypeStruct((_TOT, _EMB), jnp.float32),
      mesh=mesh,
      scratch_types=[
          pltpu.VMEM((_NCH, _CH), jnp.int32),
          pltpu.VMEM((_NBUF, _CH, _EMB), jnp.float32),
          pltpu.SemaphoreType.DMA,
      ],
      compiler_params=pltpu.CompilerParams(use_tc_tiling_on_sc=False),
  )


def _mlp_block(xn_ref, xe_ref, w1a_ref, w1b_ref, b1_ref, g1_ref, be1_ref,
               m1_ref, v1_ref, w2_ref, b2_ref, g2_ref, be2_ref, m2_ref,
               v2_ref, w3_ref, b3_ref, out_ref):
  s1 = g1_ref[...] * lax.rsqrt(v1_ref[...] + _EPS)
  t1 = be1_ref[...] - m1_ref[...] * s1
  s2 = g2_ref[...] * lax.rsqrt(v2_ref[...] + _EPS)
  t2 = be2_ref[...] - m2_ref[...] * s2

  h = jnp.dot(xn_ref[...], w1a_ref[...], preferred_element_type=jnp.float32)
  h = h + jnp.dot(xe_ref[...], w1b_ref[...],
                  preferred_element_type=jnp.float32)
  h = jnp.maximum(h + b1_ref[...], 0.0) * s1 + t1
  h = jnp.dot(h, w2_ref[...], preferred_element_type=jnp.float32)
  h = jnp.maximum(h + b2_ref[...], 0.0) * s2 + t2
  out_ref[...] = (
      jnp.dot(h, w3_ref[...], preferred_element_type=jnp.float32)
      + b3_ref[...])


def _mlp(x_num, embs, W1a, W1b, b1, g1, be1, m1, v1, W2, b2, g2, be2, m2,
         v2, W3, b3):
  grid = (_B // _BLK,)
  row = lambda i: (i, 0)
  rep = lambda i: (0, 0)
  return pl.pallas_call(
      _mlp_block,
      grid=grid,
      in_specs=[
          pl.BlockSpec((_BLK, _NUM_NUM), row),
          pl.BlockSpec((_BLK, _NUM_FIELDS * _EMB), row),
          pl.BlockSpec((_NUM_NUM, _H1), rep),
          pl.BlockSpec((_NUM_FIELDS * _EMB, _H1), rep),
          pl.BlockSpec((1, _H1), rep),
          pl.BlockSpec((1, _H1), rep),
          pl.BlockSpec((1, _H1), rep),
          pl.BlockSpec((1, _H1), rep),
          pl.BlockSpec((1, _H1), rep),
          pl.BlockSpec((_H1, _H2), rep),
          pl.BlockSpec((1, _H2), rep),
          pl.BlockSpec((1, _H2), rep),
          pl.BlockSpec((1, _H2), rep),
          pl.BlockSpec((1, _H2), rep),
          pl.BlockSpec((1, _H2), rep),
          pl.BlockSpec((_H2, 1), rep),
          pl.BlockSpec((1, 1), rep),
      ],
      out_specs=pl.BlockSpec((_BLK, 1), row),
      out_shape=jax.ShapeDtypeStruct((_B, 1), jnp.float32),
  )(x_num, embs, W1a, W1b, b1, g1, be1, m1, v1, W2, b2, g2, be2, m2, v2,
    W3, b3)


def kernel(x_num, x_cat, tables, W1, b1, g1, be1, m1, v1, W2, b2, g2, be2,
           m2, v2, W3, b3):
  tab_flat = tables.reshape(_NUM_FIELDS * _VOCAB, _EMB)
  offs = (jnp.arange(_NUM_FIELDS, dtype=jnp.int32) * _VOCAB)[None, :]
  idx2d = (x_cat + offs).reshape(_TOT // _CH, _CH)
  embs = _make_gather()(idx2d, tab_flat)
  embs = embs.reshape(_B, _NUM_FIELDS * _EMB)

  r = lambda a: a.reshape(1, -1)
  out = _mlp(x_num, embs, W1[:_NUM_NUM], W1[_NUM_NUM:], r(b1), r(g1),
             r(be1), r(m1), r(v1), W2, r(b2), r(g2), r(be2), r(m2), r(v2),
             W3, r(b3))
  return out[:, 0]
